# Initial kernel scaffold; baseline (speedup 1.0000x reference)
import jax
import jax.numpy as jnp
from jax.experimental import pallas as pl


def kernel(x, edge_index, edge_attr, batch, W1_rel, b1_rel, W1_root, W2_rel, b2_rel, W2_root, Wl1, bl1, Wl2, bl2, Wl3, bl3):
    raise NotImplementedError("write your pallas kernel here")


# speedup vs baseline: 2.4758x; 2.4758x over previous
import dataclasses
import functools

import jax
import jax.numpy as jnp
from jax import lax
from jax.experimental import pallas as pl
from jax.experimental.pallas import tpu as pltpu
from jax.experimental.pallas import tpu_sc as plsc

N = 10000
E = 320000
D = 128
H = 512
G = 64

NC = 2
NS = 16
NW = NC * NS
L = 16
B = 128

EPW = ((E // NW) + B - 1) // B * B
NB = EPW // B
E_PAD = EPW * NW

ROWS_PER_SUB = N // NS
F = 128


def _sc_edge_agg(num_chunks):
  mesh = plsc.VectorSubcoreMesh(core_axis_name="c", subcore_axis_name="s")

  def body(*refs):
    tables = refs[:num_chunks]
    src_hbm, dst_hbm, w_hbm, zeros_hbm, out_hbm = refs[num_chunks:
                                                       num_chunks + 5]
    (src_v, dst_v, w_v, rows_v, acc_sh, sem) = refs[num_chunks + 5:]

    core = lax.axis_index("c")
    sub = lax.axis_index("s")
    wid = sub * NC + core

    pltpu.sync_copy(src_hbm.at[wid], src_v)
    pltpu.sync_copy(dst_hbm.at[wid], dst_v)
    pltpu.sync_copy(w_hbm.at[wid], w_v)

    for c in range(num_chunks):
      @pl.when(sub == 0)
      def _():
        pltpu.sync_copy(zeros_hbm, acc_sh)
      plsc.subcore_barrier()

      @pl.loop(0, NB)
      def _(b):
        pltpu.async_copy(tables[c].at[src_v.at[b]], rows_v, sem).wait()

        @pl.loop(0, B)
        def _(i):
          w16 = plsc.load_gather(w_v, [jnp.full((L,), b * B + i, jnp.int32)])
          for j in range(F // L):
            sl = (i, pl.ds(j * L, L))
            rows_v[sl] = rows_v[sl] * w16

        pltpu.sync_copy(rows_v, acc_sh.at[dst_v.at[b]], add=True)

      plsc.subcore_barrier()
      @pl.when(sub == 0)
      def _():
        pltpu.sync_copy(acc_sh, out_hbm.at[c].at[core])
      plsc.subcore_barrier()

  cp = pltpu.CompilerParams()
  if "needs_layout_passes" in pltpu.CompilerParams.__dataclass_fields__:
    cp = dataclasses.replace(cp, needs_layout_passes=False)
  kern = pl.kernel(
      body,
      mesh=mesh,
      compiler_params=cp,
      out_type=jax.ShapeDtypeStruct((num_chunks, NC, N, F), jnp.float32),
      scratch_types=[
          pltpu.VMEM((NB, B), jnp.int32),
          pltpu.VMEM((NB, B), jnp.int32),
          pltpu.VMEM((EPW,), jnp.float32),
          pltpu.VMEM((B, F), jnp.float32),
          pltpu.VMEM_SHARED((N, F), jnp.float32),
          pltpu.SemaphoreType.DMA,
      ],
  )
  return kern


_sc_agg1 = _sc_edge_agg(1)
_sc_agg4 = _sc_edge_agg(4)


def _tc1_body(agg_ref, x_ref, wrel_ref, b_ref, wroot_ref, *out_refs):
  agg = agg_ref[0] + agg_ref[1]
  h = lax.dot(agg, wrel_ref[...], precision=lax.Precision.HIGHEST,
              preferred_element_type=jnp.float32)
  h += lax.dot(x_ref[...], wroot_ref[...], precision=lax.Precision.HIGHEST,
               preferred_element_type=jnp.float32)
  h = jnp.maximum(h + b_ref[...], 0.0)
  for c in range(4):
    out_refs[c][...] = h[:, c * F:(c + 1) * F]


def _tc2_body(agg_ref, h1c0, h1c1, h1c2, h1c3, batch_ref, wrel_ref, b_ref,
              wroot_ref, wl1_ref, bl1_ref, wl2_ref, bl2_ref, wl3_ref, bl3_ref,
              out_ref, pool_acc, cnt_acc):
  h1_refs = (h1c0, h1c1, h1c2, h1c3)
  i = pl.program_id(0)
  nsteps = pl.num_programs(0)

  @pl.when(i == 0)
  def _():
    pool_acc[...] = jnp.zeros_like(pool_acc)
    cnt_acc[...] = jnp.zeros_like(cnt_acc)

  h2 = jnp.zeros((BLK, H), jnp.float32) + b_ref[...]
  for c in range(4):
    a = agg_ref[2 * c] + agg_ref[2 * c + 1]
    h2 += lax.dot(a, wrel_ref[pl.ds(c * F, F), :],
                  precision=lax.Precision.HIGHEST,
                  preferred_element_type=jnp.float32)
    h2 += lax.dot(h1_refs[c][...], wroot_ref[pl.ds(c * F, F), :],
                  precision=lax.Precision.HIGHEST,
                  preferred_element_type=jnp.float32)
  h2 = jnp.maximum(h2, 0.0)

  gids = lax.broadcasted_iota(jnp.int32, (G, BLK), 0)
  bids = batch_ref[0, :, :]
  onehot = (gids == bids).astype(jnp.float32)
  pool_acc[...] += lax.dot(onehot, h2, precision=lax.Precision.HIGHEST,
                           preferred_element_type=jnp.float32)
  cnt_acc[...] += jnp.sum(onehot, axis=1, keepdims=True)

  @pl.when(i == nsteps - 1)
  def _():
    pooled = pool_acc[...] / jnp.maximum(cnt_acc[...], 1.0)
    m = jnp.maximum(lax.dot(pooled, wl1_ref[...],
                            precision=lax.Precision.HIGHEST,
                            preferred_element_type=jnp.float32)
                    + bl1_ref[...], 0.0)
    m = jnp.maximum(lax.dot(m, wl2_ref[...],
                            precision=lax.Precision.HIGHEST,
                            preferred_element_type=jnp.float32)
                    + bl2_ref[...], 0.0)
    out_ref[...] = (jnp.sum(m * wl3_ref[...], axis=1, keepdims=True)
                    + bl3_ref[...])


BLK = 1000


def kernel(x, edge_index, edge_attr, batch,
           W1_rel, b1_rel, W1_root, W2_rel, b2_rel, W2_root,
           Wl1, bl1, Wl2, bl2, Wl3, bl3):
  src = edge_index[0]
  dst = edge_index[1]
  pad = E_PAD - E
  src_p = jnp.concatenate([src, jnp.zeros((pad,), jnp.int32)]).reshape(
      NW, NB, B)
  dst_p = jnp.concatenate([dst, jnp.zeros((pad,), jnp.int32)]).reshape(
      NW, NB, B)
  w_p = jnp.concatenate([edge_attr, jnp.zeros((pad,), jnp.float32)]).reshape(
      NW, EPW)

  zeros_nf = jnp.zeros((N, F), jnp.float32)

  agg1 = _sc_agg1(x, src_p, dst_p, w_p, zeros_nf)

  ngrid = N // BLK
  h1c = pl.pallas_call(
      _tc1_body,
      grid=(ngrid,),
      in_specs=[
          pl.BlockSpec((2, BLK, D), lambda i: (0, i, 0)),
          pl.BlockSpec((BLK, D), lambda i: (i, 0)),
          pl.BlockSpec((D, H), lambda i: (0, 0)),
          pl.BlockSpec((1, H), lambda i: (0, 0)),
          pl.BlockSpec((D, H), lambda i: (0, 0)),
      ],
      out_specs=[pl.BlockSpec((BLK, F), lambda i: (i, 0))] * 4,
      out_shape=[jax.ShapeDtypeStruct((N, F), jnp.float32)] * 4,
  )(agg1[0], x, W1_rel.T, b1_rel.reshape(1, H), W1_root.T)

  agg2 = _sc_agg4(h1c[0], h1c[1], h1c[2], h1c[3], src_p, dst_p, w_p,
                  zeros_nf)
  agg2 = agg2.reshape(8, N, F)

  out = pl.pallas_call(
      _tc2_body,
      grid=(ngrid,),
      in_specs=[
          pl.BlockSpec((8, BLK, F), lambda i: (0, i, 0)),
          pl.BlockSpec((BLK, F), lambda i: (i, 0)),
          pl.BlockSpec((BLK, F), lambda i: (i, 0)),
          pl.BlockSpec((BLK, F), lambda i: (i, 0)),
          pl.BlockSpec((BLK, F), lambda i: (i, 0)),
          pl.BlockSpec((1, 1, BLK), lambda i: (i, 0, 0)),
          pl.BlockSpec((H, H), lambda i: (0, 0)),
          pl.BlockSpec((1, H), lambda i: (0, 0)),
          pl.BlockSpec((H, H), lambda i: (0, 0)),
          pl.BlockSpec((H, G), lambda i: (0, 0)),
          pl.BlockSpec((1, G), lambda i: (0, 0)),
          pl.BlockSpec((G, 16), lambda i: (0, 0)),
          pl.BlockSpec((1, 16), lambda i: (0, 0)),
          pl.BlockSpec((1, 16), lambda i: (0, 0)),
          pl.BlockSpec((1, 1), lambda i: (0, 0)),
      ],
      out_specs=pl.BlockSpec((G, 1), lambda i: (0, 0)),
      out_shape=jax.ShapeDtypeStruct((G, 1), jnp.float32),
      scratch_shapes=[
          pltpu.VMEM((G, H), jnp.float32),
          pltpu.VMEM((G, 1), jnp.float32),
      ],
  )(agg2,
    h1c[0], h1c[1], h1c[2], h1c[3],
    batch.reshape(ngrid, 1, BLK),
    W2_rel.T, b2_rel.reshape(1, H), W2_root.T,
    Wl1.T, bl1.reshape(1, G), Wl2.T, bl2.reshape(1, 16),
    Wl3, bl3.reshape(1, 1))
  return out
